# baseline (speedup 1.0000x reference)
import functools

import jax
import jax.numpy as jnp
from jax import lax
from jax.experimental import pallas as pl
from jax.experimental.pallas import tpu as pltpu
from jax.experimental.pallas import tpu_sc as plsc

N_NODES = 10000
N_FEATS = 128
N_HEADS = 8
D_HEAD = 16
N_EDGES = 320000
ALPHA = 0.2

NC = 2
NS = 16
NW = NC * NS
CHUNK = 128
NCHUNK = 80
EPW = NCHUNK * CHUNK
PADE = NW * EPW - N_EDGES
NRING = 2
NGRP = NCHUNK // NRING - 1
N_PAD = 10240
RPT = N_PAD // NS

BLK = 1000
GRID = N_NODES // BLK


def _dense_body(x_ref, wf_ref, aa_ref, wh_ref, ee_ref, eer_ref, mx_ref,
                cv_ref):
    i = pl.program_id(0)
    wh = jnp.dot(x_ref[...], wf_ref[...], preferred_element_type=jnp.float32)
    wh_ref[...] = wh.astype(jnp.bfloat16)
    ee = jnp.dot(wh, aa_ref[...], preferred_element_type=jnp.float32)
    ee_ref[...] = ee
    eer_ref[...] = jnp.concatenate(
        [ee[:, N_HEADS:2 * N_HEADS], ee[:, 0:N_HEADS]], axis=1)
    rm = jnp.max(ee, axis=0, keepdims=True)

    @pl.when(i == 0)
    def _():
        mx_ref[...] = rm

    @pl.when(i > 0)
    def _():
        mx_ref[...] = jnp.maximum(mx_ref[...], rm)

    @pl.when(i == pl.num_programs(0) - 1)
    def _():
        m = mx_ref[...]
        s = m[:, 0:N_HEADS] + m[:, N_HEADS:2 * N_HEADS]
        cv_ref[:, 0:N_HEADS] = jnp.maximum(s, ALPHA * s)
        cv_ref[:, N_HEADS:2 * N_HEADS] = jnp.full((1, N_HEADS), 1e9,
                                                  jnp.float32)


def _phase_a(x, wf, aa):
    return pl.pallas_call(
        _dense_body,
        grid=(GRID,),
        in_specs=[
            pl.BlockSpec((BLK, N_FEATS), lambda i: (i, 0)),
            pl.BlockSpec((N_FEATS, N_FEATS), lambda i: (0, 0)),
            pl.BlockSpec((N_FEATS, 2 * N_HEADS), lambda i: (0, 0)),
        ],
        out_specs=[
            pl.BlockSpec((BLK, N_FEATS), lambda i: (i, 0)),
            pl.BlockSpec((BLK, 2 * N_HEADS), lambda i: (i, 0)),
            pl.BlockSpec((BLK, 2 * N_HEADS), lambda i: (i, 0)),
            pl.BlockSpec((1, 2 * N_HEADS), lambda i: (0, 0)),
            pl.BlockSpec((1, 2 * N_HEADS), lambda i: (0, 0)),
        ],
        out_shape=[
            jax.ShapeDtypeStruct((N_NODES, N_FEATS), jnp.bfloat16),
            jax.ShapeDtypeStruct((N_NODES, 2 * N_HEADS), jnp.float32),
            jax.ShapeDtypeStruct((N_NODES, 2 * N_HEADS), jnp.float32),
            jax.ShapeDtypeStruct((1, 2 * N_HEADS), jnp.float32),
            jax.ShapeDtypeStruct((1, 2 * N_HEADS), jnp.float32),
        ],
    )(x, wf, aa)


def _sc_body(ee_hbm, eer_hbm, wh_hbm, src_hbm, dst_hbm, cv_hbm,
             acc_hbm, den_hbm, *scr):
    sb = scr[0:NRING]
    db = scr[NRING:2 * NRING]
    es = scr[2 * NRING:3 * NRING]
    ed = scr[3 * NRING:4 * NRING]
    wb = scr[4 * NRING:5 * NRING]
    dsc, whs, ex_v, cv_v, acc_sh, den_sh = scr[5 * NRING:5 * NRING + 6]
    semg = scr[5 * NRING + 6:6 * NRING + 6]
    semi = scr[6 * NRING + 6:7 * NRING + 6]
    c = lax.axis_index("c")
    s = lax.axis_index("s")
    wid = c * NS + s
    r0 = s * RPT

    @pl.loop(0, CHUNK)
    def _zb_fill(r):
        for g in range(N_FEATS // 16):
            whs[r, pl.ds(g * 16, 16)] = jnp.zeros((16,), jnp.float32)

    lane = jnp.arange(16, dtype=jnp.int32)
    lo8 = lane < 8
    rowpair = lane >> 3

    @pl.loop(0, CHUNK // 2)
    def _zden_fill(r):
        plsc.store_scatter(ex_v, [2 * r + rowpair, lane & 7],
                           jnp.zeros((16,), jnp.float32))

    @pl.loop(0, RPT // CHUNK)
    def _zcopy(k):
        pltpu.sync_copy(whs, acc_sh.at[pl.ds(r0 + k * CHUNK, CHUNK)])
        pltpu.sync_copy(ex_v, den_sh.at[pl.ds(r0 + k * CHUNK, CHUNK)])

    plsc.subcore_barrier()

    pltpu.sync_copy(cv_hbm, cv_v)
    cvec = cv_v[...]
    hp = jnp.arange(16, dtype=jnp.int32)
    hsel = [2 * g + (hp >> 3) for g in range(N_FEATS // 32)]
    hmask = jnp.int32(-65536)

    def issue_i(cidx, k):
        base = wid * EPW + cidx * CHUNK
        pltpu.async_copy(src_hbm.at[pl.ds(base, CHUNK)], sb[k], semi[k])
        pltpu.async_copy(dst_hbm.at[pl.ds(base, CHUNK)], db[k], semi[k])

    def drain_i(k):
        pltpu.make_async_copy(src_hbm.at[pl.ds(0, CHUNK)], sb[k],
                              semi[k]).wait()
        pltpu.make_async_copy(dst_hbm.at[pl.ds(0, CHUNK)], db[k],
                              semi[k]).wait()

    def issue_g(k):
        pltpu.async_copy(ee_hbm.at[sb[k]], es[k], semg[k])
        pltpu.async_copy(eer_hbm.at[db[k]], ed[k], semg[k])
        pltpu.async_copy(wh_hbm.at[sb[k]], wb[k], semg[k])

    def drain_g(k):
        pltpu.make_async_copy(ee_hbm.at[pl.ds(0, CHUNK)], es[k],
                              semg[k]).wait()
        pltpu.make_async_copy(eer_hbm.at[pl.ds(0, CHUNK)], ed[k],
                              semg[k]).wait()
        pltpu.make_async_copy(wh_hbm.at[pl.ds(0, CHUNK)], wb[k],
                              semg[k]).wait()

    def save_didx(k):
        for g in range(CHUNK // 16):
            dsc[pl.ds(g * 16, 16)] = db[k][pl.ds(g * 16, 16)]

    def compute(k):
        eesb, eedb, whb = es[k], ed[k], wb[k]

        @plsc.parallel_loop(0, CHUNK, unroll=4)
        def _edge(b):
            bb = jnp.full((16,), b, jnp.int32)
            sv = eesb[b, :] + eedb[b, :]
            sv = jnp.maximum(sv, ALPHA * sv)
            ex = jnp.exp(sv - cvec)
            plsc.store_scatter(ex_v, [bb, lane & 7], ex, mask=lo8)
            for g in range(N_FEATS // 32):
                bh = ex.at[hsel[g]].get(mode='promise_in_bounds')
                v = whb[b, pl.ds(g * 16, 16)]
                flo = plsc.bitcast(v << 16, jnp.float32)
                fhi = plsc.bitcast(v & hmask, jnp.float32)
                whs[b, pl.ds(g * 32, 16)] = flo * bh
                whs[b, pl.ds(g * 32 + 16, 16)] = fhi * bh

        pltpu.sync_copy(ex_v, den_sh.at[dsc], add=True)
        pltpu.sync_copy(whs, acc_sh.at[dsc], add=True)

    for k in range(NRING):
        issue_i(k, k)
    for k in range(NRING - 1):
        drain_i(k)
        issue_g(k)

    @pl.loop(0, NGRP)
    def _grp(p):
        c0 = NRING * p
        for k in range(NRING):
            kn = (k + NRING - 1) % NRING
            drain_g(k)
            save_didx(k)
            drain_i(kn)
            issue_g(kn)
            issue_i(c0 + k + NRING, k)
            compute(k)

    for k in range(NRING):
        drain_g(k)
        save_didx(k)
        if k == 0:
            drain_i(NRING - 1)
            issue_g(NRING - 1)
        compute(k)

    plsc.subcore_barrier()
    pltpu.sync_copy(acc_sh.at[pl.ds(r0, RPT)], acc_hbm.at[c].at[pl.ds(r0, RPT)])
    pltpu.sync_copy(den_sh.at[pl.ds(r0, RPT)], den_hbm.at[c].at[pl.ds(r0, RPT)])


def _phase_b(ee, eer, wh, src2, dst2, cv):
    mesh = plsc.VectorSubcoreMesh(core_axis_name="c", subcore_axis_name="s",
                                  num_cores=NC, num_subcores=NS)
    f = pl.kernel(
        _sc_body,
        out_type=[
            jax.ShapeDtypeStruct((NC, N_PAD, N_FEATS), jnp.float32),
            jax.ShapeDtypeStruct((NC, N_PAD, N_HEADS), jnp.float32),
        ],
        mesh=mesh,
        scratch_types=(
            [pltpu.VMEM((CHUNK,), jnp.int32)] * (2 * NRING)
            + [pltpu.VMEM((CHUNK, 2 * N_HEADS), jnp.float32)] * (2 * NRING)
            + [pltpu.VMEM((CHUNK, N_FEATS // 2), jnp.int32)] * NRING
            + [
                pltpu.VMEM((CHUNK,), jnp.int32),
                pltpu.VMEM((CHUNK, N_FEATS), jnp.float32),
                pltpu.VMEM((CHUNK, N_HEADS), jnp.float32),
                pltpu.VMEM((16,), jnp.float32),
                pltpu.VMEM_SHARED((N_PAD, N_FEATS), jnp.float32),
                pltpu.VMEM_SHARED((N_PAD, N_HEADS), jnp.float32),
            ]
            + [pltpu.SemaphoreType.DMA] * (2 * NRING)
        ),
        compiler_params=pltpu.CompilerParams(needs_layout_passes=False,
                                             use_tc_tiling_on_sc=False),
    )
    return f(ee, eer, wh, src2, dst2, cv)


def _norm_body(a0_ref, a1_ref, d0_ref, d1_ref, e8_ref, pm_ref, o_ref):
    d = d0_ref[0] + d1_ref[0]
    scale = 1.0 / jnp.maximum(d, 1e-16)
    sc = jnp.dot(scale, e8_ref[...], preferred_element_type=jnp.float32)
    o_ref[...] = jnp.dot((a0_ref[0] + a1_ref[0]) * sc, pm_ref[...],
                         preferred_element_type=jnp.float32)


def _phase_c(acc, den, e8, pm):
    return pl.pallas_call(
        _norm_body,
        grid=(GRID,),
        in_specs=[
            pl.BlockSpec((1, BLK, N_FEATS), lambda i: (0, i, 0)),
            pl.BlockSpec((1, BLK, N_FEATS), lambda i: (1, i, 0)),
            pl.BlockSpec((1, BLK, N_HEADS), lambda i: (0, i, 0)),
            pl.BlockSpec((1, BLK, N_HEADS), lambda i: (1, i, 0)),
            pl.BlockSpec((N_HEADS, N_FEATS), lambda i: (0, 0)),
            pl.BlockSpec((N_FEATS, N_FEATS), lambda i: (0, 0)),
        ],
        out_specs=pl.BlockSpec((BLK, N_FEATS), lambda i: (i, 0)),
        out_shape=jax.ShapeDtypeStruct((N_NODES, N_FEATS), jnp.float32),
    )(acc, acc, den, den, e8, pm)


def kernel(x, edge_index, W, a):
    wf = jnp.transpose(W, (1, 0, 2)).reshape(N_FEATS, N_FEATS)
    a1 = a[:, :D_HEAD, 0]
    a2 = a[:, D_HEAD:, 0]
    eye = jnp.eye(N_HEADS, dtype=jnp.float32)
    aa = jnp.concatenate([
        jnp.einsum('ho,hk->hok', a1, eye).reshape(N_FEATS, N_HEADS),
        jnp.einsum('ho,hk->hok', a2, eye).reshape(N_FEATS, N_HEADS),
    ], axis=1)
    wh16, ee, eer, _mx, cv = _phase_a(x, wf, aa)
    wh_pk = jax.lax.bitcast_convert_type(
        wh16.reshape(N_NODES, N_FEATS // 2, 2), jnp.int32)

    pad_src = jnp.zeros((PADE,), jnp.int32)
    pad_dst = N_NODES + (jnp.arange(PADE, dtype=jnp.int32)
                         % (N_PAD - N_NODES))
    src_p = jnp.concatenate([edge_index[0].astype(jnp.int32), pad_src])
    dst_p = jnp.concatenate([edge_index[1].astype(jnp.int32), pad_dst])
    eer_p = jnp.concatenate(
        [eer, jnp.zeros((N_PAD - N_NODES, 2 * N_HEADS), jnp.float32)])

    acc, den = _phase_b(ee, eer_p, wh_pk, src_p, dst_p,
                        cv.reshape(2 * N_HEADS))

    cols = jnp.arange(N_FEATS)
    g = cols // 32
    r = cols % 32
    orig = 32 * g + jnp.where(r < 16, 2 * r, 2 * (r - 16) + 1)
    pm = (orig[:, None] == cols[None, :]).astype(jnp.float32)
    e8p = (jnp.arange(N_HEADS)[:, None] == (orig // D_HEAD)[None, :]
           ).astype(jnp.float32)
    return _phase_c(acc, den, e8p, pm)


# speedup vs baseline: 1.6356x; 1.6356x over previous
import functools

import jax
import jax.numpy as jnp
from jax import lax
from jax.experimental import pallas as pl
from jax.experimental.pallas import tpu as pltpu
from jax.experimental.pallas import tpu_sc as plsc

N_NODES = 10000
N_FEATS = 128
N_HEADS = 8
D_HEAD = 16
N_EDGES = 320000
ALPHA = 0.2

NC = 2
NS = 16
NW = NC * NS
EPW = N_EDGES // NW
CHUNK = 80
NCHUNK = EPW // CHUNK
NPAIR = (NCHUNK - 1) // 2
N_PAD = 10240
RPT = N_PAD // NS
ZROWS = 128

BLK = 1000
GRID = N_NODES // BLK


def _dense_body(x_ref, wf_ref, aa_ref, wh_ref, ee_ref, eer_ref, mx_ref,
                cv_ref):
    i = pl.program_id(0)
    wh = jnp.dot(x_ref[...], wf_ref[...], preferred_element_type=jnp.float32)
    wh_ref[...] = wh.astype(jnp.bfloat16)
    ee = jnp.dot(wh, aa_ref[...], preferred_element_type=jnp.float32)
    ee_ref[...] = ee
    eer_ref[...] = jnp.concatenate(
        [ee[:, N_HEADS:2 * N_HEADS], ee[:, 0:N_HEADS]], axis=1)
    rm = jnp.max(ee, axis=0, keepdims=True)

    @pl.when(i == 0)
    def _():
        mx_ref[...] = rm

    @pl.when(i > 0)
    def _():
        mx_ref[...] = jnp.maximum(mx_ref[...], rm)

    @pl.when(i == pl.num_programs(0) - 1)
    def _():
        m = mx_ref[...]
        s = m[:, 0:N_HEADS] + m[:, N_HEADS:2 * N_HEADS]
        cv_ref[:, 0:N_HEADS] = jnp.maximum(s, ALPHA * s)
        cv_ref[:, N_HEADS:2 * N_HEADS] = jnp.full((1, N_HEADS), 1e9,
                                                  jnp.float32)


def _phase_a(x, wf, aa):
    return pl.pallas_call(
        _dense_body,
        grid=(GRID,),
        in_specs=[
            pl.BlockSpec((BLK, N_FEATS), lambda i: (i, 0)),
            pl.BlockSpec((N_FEATS, N_FEATS), lambda i: (0, 0)),
            pl.BlockSpec((N_FEATS, 2 * N_HEADS), lambda i: (0, 0)),
        ],
        out_specs=[
            pl.BlockSpec((BLK, N_FEATS), lambda i: (i, 0)),
            pl.BlockSpec((BLK, 2 * N_HEADS), lambda i: (i, 0)),
            pl.BlockSpec((BLK, 2 * N_HEADS), lambda i: (i, 0)),
            pl.BlockSpec((1, 2 * N_HEADS), lambda i: (0, 0)),
            pl.BlockSpec((1, 2 * N_HEADS), lambda i: (0, 0)),
        ],
        out_shape=[
            jax.ShapeDtypeStruct((N_NODES, N_FEATS), jnp.bfloat16),
            jax.ShapeDtypeStruct((N_NODES, 2 * N_HEADS), jnp.float32),
            jax.ShapeDtypeStruct((N_NODES, 2 * N_HEADS), jnp.float32),
            jax.ShapeDtypeStruct((1, 2 * N_HEADS), jnp.float32),
            jax.ShapeDtypeStruct((1, 2 * N_HEADS), jnp.float32),
        ],
    )(x, wf, aa)


def _sc_body(ee_hbm, eer_hbm, wh_hbm, src_hbm, dst_hbm, cv_hbm,
             acc_hbm, den_hbm,
             sbuf0, dbuf0, sbuf1, dbuf1, dsc, ee_s0, ee_d0, whr0,
             ee_s1, ee_d1, whr1, whs,
             ex_v, cv_v,
             acc_sh, den_sh, semg0, semg1, semi0, semi1):
    c = lax.axis_index("c")
    s = lax.axis_index("s")
    wid = c * NS + s
    r0 = s * RPT

    @pl.loop(0, CHUNK)
    def _zb_fill(r):
        for g in range(N_FEATS // 16):
            whs[r, pl.ds(g * 16, 16)] = jnp.zeros((16,), jnp.float32)

    lane = jnp.arange(16, dtype=jnp.int32)
    lo8 = lane < 8
    rowpair = lane >> 3

    @pl.loop(0, CHUNK // 2)
    def _zden_fill(r):
        plsc.store_scatter(ex_v, [2 * r + rowpair, lane & 7],
                           jnp.zeros((16,), jnp.float32))

    @pl.loop(0, RPT // CHUNK)
    def _zcopy(k):
        pltpu.sync_copy(whs, acc_sh.at[pl.ds(r0 + k * CHUNK, CHUNK)])
        pltpu.sync_copy(ex_v, den_sh.at[pl.ds(r0 + k * CHUNK, CHUNK)])

    plsc.subcore_barrier()

    pltpu.sync_copy(cv_hbm, cv_v)
    cvec = cv_v[...]
    hp = jnp.arange(16, dtype=jnp.int32)
    hsel = [2 * g + (hp >> 3) for g in range(N_FEATS // 32)]
    hmask = jnp.int32(-65536)

    def issue_i(cidx, sb, db, sem):
        base = wid * EPW + cidx * CHUNK
        pltpu.async_copy(src_hbm.at[pl.ds(base, CHUNK)], sb, sem)
        pltpu.async_copy(dst_hbm.at[pl.ds(base, CHUNK)], db, sem)

    def drain_i(sb, db, sem):
        pltpu.make_async_copy(src_hbm.at[pl.ds(0, CHUNK)], sb, sem).wait()
        pltpu.make_async_copy(dst_hbm.at[pl.ds(0, CHUNK)], db, sem).wait()

    def issue_g(sb, db, eesb, eedb, whb, sem):
        pltpu.async_copy(ee_hbm.at[sb], eesb, sem)
        pltpu.async_copy(eer_hbm.at[db], eedb, sem)
        pltpu.async_copy(wh_hbm.at[sb], whb, sem)

    def drain_g(eesb, eedb, whb, sem):
        pltpu.make_async_copy(ee_hbm.at[pl.ds(0, CHUNK)], eesb, sem).wait()
        pltpu.make_async_copy(eer_hbm.at[pl.ds(0, CHUNK)], eedb, sem).wait()
        pltpu.make_async_copy(wh_hbm.at[pl.ds(0, CHUNK)], whb, sem).wait()

    def save_didx(db):
        for g in range(CHUNK // 16):
            dsc[pl.ds(g * 16, 16)] = db[pl.ds(g * 16, 16)]

    def compute(db, eesb, eedb, whb):
        @plsc.parallel_loop(0, CHUNK, unroll=4)
        def _edge(b):
            bb = jnp.full((16,), b, jnp.int32)
            sv = eesb[b, :] + eedb[b, :]
            sv = jnp.maximum(sv, ALPHA * sv)
            ex = jnp.exp(sv - cvec)
            plsc.store_scatter(ex_v, [bb, lane & 7], ex, mask=lo8)
            for g in range(N_FEATS // 32):
                bh = ex.at[hsel[g]].get(mode='promise_in_bounds')
                v = whb[b, pl.ds(g * 16, 16)]
                flo = plsc.bitcast(v << 16, jnp.float32)
                fhi = plsc.bitcast(v & hmask, jnp.float32)
                whs[b, pl.ds(g * 32, 16)] = flo * bh
                whs[b, pl.ds(g * 32 + 16, 16)] = fhi * bh

        pltpu.sync_copy(ex_v, den_sh.at[db], add=True)
        pltpu.sync_copy(whs, acc_sh.at[db], add=True)

    issue_i(0, sbuf0, dbuf0, semi0)
    issue_i(1, sbuf1, dbuf1, semi1)
    drain_i(sbuf0, dbuf0, semi0)
    issue_g(sbuf0, dbuf0, ee_s0, ee_d0, whr0, semg0)

    @pl.loop(0, NPAIR)
    def _pair(p):
        drain_g(ee_s0, ee_d0, whr0, semg0)
        save_didx(dbuf0)
        drain_i(sbuf1, dbuf1, semi1)
        issue_g(sbuf1, dbuf1, ee_s1, ee_d1, whr1, semg1)
        issue_i(2 * p + 2, sbuf0, dbuf0, semi0)
        compute(dsc, ee_s0, ee_d0, whr0)
        drain_g(ee_s1, ee_d1, whr1, semg1)
        save_didx(dbuf1)
        drain_i(sbuf0, dbuf0, semi0)
        issue_g(sbuf0, dbuf0, ee_s0, ee_d0, whr0, semg0)
        issue_i(jnp.minimum(2 * p + 3, NCHUNK - 1), sbuf1, dbuf1, semi1)
        compute(dsc, ee_s1, ee_d1, whr1)

    drain_g(ee_s0, ee_d0, whr0, semg0)
    drain_i(sbuf1, dbuf1, semi1)
    compute(dbuf0, ee_s0, ee_d0, whr0)

    plsc.subcore_barrier()
    pltpu.sync_copy(acc_sh.at[pl.ds(r0, RPT)], acc_hbm.at[c].at[pl.ds(r0, RPT)])
    pltpu.sync_copy(den_sh.at[pl.ds(r0, RPT)], den_hbm.at[c].at[pl.ds(r0, RPT)])


def _phase_b(ee, eer, wh, src2, dst2, cv):
    mesh = plsc.VectorSubcoreMesh(core_axis_name="c", subcore_axis_name="s",
                                  num_cores=NC, num_subcores=NS)
    f = pl.kernel(
        _sc_body,
        out_type=[
            jax.ShapeDtypeStruct((NC, N_PAD, N_FEATS), jnp.float32),
            jax.ShapeDtypeStruct((NC, N_PAD, N_HEADS), jnp.float32),
        ],
        mesh=mesh,
        scratch_types=[
            pltpu.VMEM((CHUNK,), jnp.int32),
            pltpu.VMEM((CHUNK,), jnp.int32),
            pltpu.VMEM((CHUNK,), jnp.int32),
            pltpu.VMEM((CHUNK,), jnp.int32),
            pltpu.VMEM((CHUNK,), jnp.int32),
            pltpu.VMEM((CHUNK, 2 * N_HEADS), jnp.float32),
            pltpu.VMEM((CHUNK, 2 * N_HEADS), jnp.float32),
            pltpu.VMEM((CHUNK, N_FEATS // 2), jnp.int32),
            pltpu.VMEM((CHUNK, 2 * N_HEADS), jnp.float32),
            pltpu.VMEM((CHUNK, 2 * N_HEADS), jnp.float32),
            pltpu.VMEM((CHUNK, N_FEATS // 2), jnp.int32),
            pltpu.VMEM((CHUNK, N_FEATS), jnp.float32),
            pltpu.VMEM((CHUNK, N_HEADS), jnp.float32),
            pltpu.VMEM((16,), jnp.float32),
            pltpu.VMEM_SHARED((N_PAD, N_FEATS), jnp.float32),
            pltpu.VMEM_SHARED((N_PAD, N_HEADS), jnp.float32),
            pltpu.SemaphoreType.DMA,
            pltpu.SemaphoreType.DMA,
            pltpu.SemaphoreType.DMA,
            pltpu.SemaphoreType.DMA,
        ],
        compiler_params=pltpu.CompilerParams(needs_layout_passes=False,
                                             use_tc_tiling_on_sc=False),
    )
    return f(ee, eer, wh, src2, dst2, cv)


def _norm_body(a0_ref, a1_ref, d0_ref, d1_ref, e8_ref, pm_ref, o_ref):
    d = d0_ref[0] + d1_ref[0]
    scale = 1.0 / jnp.maximum(d, 1e-16)
    sc = jnp.dot(scale, e8_ref[...], preferred_element_type=jnp.float32)
    o_ref[...] = jnp.dot((a0_ref[0] + a1_ref[0]) * sc, pm_ref[...],
                         preferred_element_type=jnp.float32)


def _phase_c(acc, den, e8, pm):
    return pl.pallas_call(
        _norm_body,
        grid=(GRID,),
        in_specs=[
            pl.BlockSpec((1, BLK, N_FEATS), lambda i: (0, i, 0)),
            pl.BlockSpec((1, BLK, N_FEATS), lambda i: (1, i, 0)),
            pl.BlockSpec((1, BLK, N_HEADS), lambda i: (0, i, 0)),
            pl.BlockSpec((1, BLK, N_HEADS), lambda i: (1, i, 0)),
            pl.BlockSpec((N_HEADS, N_FEATS), lambda i: (0, 0)),
            pl.BlockSpec((N_FEATS, N_FEATS), lambda i: (0, 0)),
        ],
        out_specs=pl.BlockSpec((BLK, N_FEATS), lambda i: (i, 0)),
        out_shape=jax.ShapeDtypeStruct((N_NODES, N_FEATS), jnp.float32),
    )(acc, acc, den, den, e8, pm)


def kernel(x, edge_index, W, a):
    wf = jnp.transpose(W, (1, 0, 2)).reshape(N_FEATS, N_FEATS)
    a1 = a[:, :D_HEAD, 0]
    a2 = a[:, D_HEAD:, 0]
    eye = jnp.eye(N_HEADS, dtype=jnp.float32)
    aa = jnp.concatenate([
        jnp.einsum('ho,hk->hok', a1, eye).reshape(N_FEATS, N_HEADS),
        jnp.einsum('ho,hk->hok', a2, eye).reshape(N_FEATS, N_HEADS),
    ], axis=1)
    wh16, ee, eer, _mx, cv = _phase_a(x, wf, aa)
    wh_pk = jax.lax.bitcast_convert_type(
        wh16.reshape(N_NODES, N_FEATS // 2, 2), jnp.int32)

    acc, den = _phase_b(ee, eer, wh_pk, edge_index[0], edge_index[1],
                        cv.reshape(2 * N_HEADS))

    cols = jnp.arange(N_FEATS)
    g = cols // 32
    r = cols % 32
    orig = 32 * g + jnp.where(r < 16, 2 * r, 2 * (r - 16) + 1)
    pm = (orig[:, None] == cols[None, :]).astype(jnp.float32)
    e8p = (jnp.arange(N_HEADS)[:, None] == (orig // D_HEAD)[None, :]
           ).astype(jnp.float32)
    return _phase_c(acc, den, e8p, pm)
